# baseline (speedup 1.0000x reference)
import functools

import jax
import jax.numpy as jnp
from jax import lax
from jax.experimental import pallas as pl
from jax.experimental.pallas import tpu as pltpu
from jax.experimental.pallas import tpu_sc as plsc

N_TOTAL = 50000
N_ATTR = 10000
N_UN = N_TOTAL - N_ATTR
D_IN = 512
D_HID = 256
K = 8

TILE_D = 2000
NC, NS, L = 2, 16, 16
CHUNK = 1280
N_UN_PAD = NC * NS * CHUNK
HALF = NS * CHUNK
GSUB = 128
NVR = CHUNK // L
TILE_M = 4096
CAP = NC * HALF
CBUF_ROWS = 45056


def _elu(x):
    return jnp.where(x > 0, x, jnp.exp(jnp.minimum(x, 0.0)) - 1.0)


def _rup128(c):
    return ((c + GSUB - 1) // GSUB) * GSUB


def _table_body(bops_ref, w1_ref, b1_ref, w2_ref, b2_ref, out_ref):
    c = jnp.concatenate(
        [jnp.zeros((1, D_HID), jnp.float32), _elu(bops_ref[...])], axis=0)
    t = _elu(jnp.dot(c, w1_ref[...], preferred_element_type=jnp.float32)
             + b1_ref[...])
    res = _elu(jnp.dot(t, w2_ref[...], preferred_element_type=jnp.float32)
               + b2_ref[...])
    out_ref[...] = _elu(c + res)


def _dense_body(a_ref, x_ref, tbl_ref, wpre_ref, bpre_ref, wops_ref,
                bops_ref, w1_ref, b1_ref, w2_ref, b2_ref, out_ref):
    i = pl.program_id(0)
    a = a_ref[0, 0, :][:, None]

    @pl.when(i < N_ATTR // TILE_D)
    def _():
        x = x_ref[...]
        h_tr = jnp.dot(x, wpre_ref[...], preferred_element_type=jnp.float32)
        h_tr = h_tr + bpre_ref[...]
        acc = jnp.zeros((TILE_D, D_HID), dtype=jnp.float32)
        for k in range(1, K):
            o = jnp.dot(h_tr, wops_ref[k - 1],
                        preferred_element_type=jnp.float32)
            o = _elu(o + bops_ref[k - 1][None, :])
            acc = acc + jnp.where(a == k, o, 0.0)
        t = _elu(jnp.dot(acc, w1_ref[...], preferred_element_type=jnp.float32)
                 + b1_ref[...])
        res = _elu(jnp.dot(t, w2_ref[...], preferred_element_type=jnp.float32)
                   + b2_ref[...])
        out_ref[...] = _elu(acc + res) + h_tr

    @pl.when(i >= N_ATTR // TILE_D)
    def _():
        acc = jnp.zeros((TILE_D, D_HID), dtype=jnp.float32)
        for k in range(K):
            acc = acc + jnp.where(a == k, tbl_ref[k][None, :], 0.0)
        out_ref[...] = acc


def _sc_route_body(a_hbm, emb_hbm, gath_hbm, jidx_hbm, cnts_hbm, tot_hbm,
                   half_v, jcomp_v, rows_v, cbuf_v, sem):
    cid = lax.axis_index("c")
    sid = lax.axis_index("s")
    hbase = pl.multiple_of(cid * HALF, 8)
    pltpu.sync_copy(a_hbm.at[pl.ds(hbase, HALF)], half_v.at[pl.ds(0, HALF)])
    lane = lax.broadcasted_iota(jnp.int32, (L,), 0)

    cnt_t = []
    for t in range(NS):
        def step(v, acc):
            av = half_v[pl.ds(v * L, L)]
            return acc + (av == 0).astype(jnp.int32)
        acc = lax.fori_loop(t * NVR, (t + 1) * NVR, step,
                            jnp.zeros((L,), jnp.int32))
        cnt_t.append(jnp.cumsum(acc)[L - 1])

    off = cid * HALF
    cnt = jnp.int32(0)
    tot = jnp.int32(0)
    for t in range(NS):
        off = off + jnp.where(t < sid, _rup128(cnt_t[t]), 0)
        cnt = cnt + jnp.where(t == sid, cnt_t[t], 0)
        tot = tot + _rup128(cnt_t[t])
    base = cid * HALF + sid * CHUNK

    cbuf_v[...] = (lane == 0).astype(jnp.int32) * cnt
    w16 = pl.multiple_of((cid * NS + sid) * L, 8)
    pltpu.sync_copy(cbuf_v, cnts_hbm.at[pl.ds(w16, L)])

    @pl.when(sid == 0)
    def _():
        cbuf_v[...] = (lane == 0).astype(jnp.int32) * tot
        pltpu.sync_copy(cbuf_v, tot_hbm.at[pl.ds(pl.multiple_of(cid * L, 8), L)])

    zero16 = jnp.zeros((L,), jnp.int32)
    for v in range(NVR):
        jcomp_v[pl.ds(v * L, L)] = zero16
    run = jnp.int32(0)
    for v in range(NVR):
        av = half_v[pl.ds(sid * CHUNK + v * L, L)]
        m = av == 0
        mi = m.astype(jnp.int32)
        inc = jnp.cumsum(mi)
        jvec = base + v * L + lane
        plsc.store_scatter(jcomp_v, [run + inc - mi], jvec, mask=m)
        run = run + inc[L - 1]

    @pl.when(cnt > 0)
    def _():
        lastv = jcomp_v[pl.ds(cnt - 1, L)][0]
        for v in range(NVR):
            cv = jcomp_v[pl.ds(v * L, L)]
            g = v * L + lane
            jcomp_v[pl.ds(v * L, L)] = jnp.where(g < cnt, cv, lastv)

    nwin = (cnt + GSUB - 1) // GSUB

    def win(i, _):
        pltpu.async_copy(
            emb_hbm.at[jcomp_v.at[pl.ds(i * GSUB, GSUB)]], rows_v, sem
        ).wait()
        dst0 = pl.multiple_of(off + i * GSUB, 8)
        pltpu.sync_copy(rows_v, gath_hbm.at[pl.ds(dst0, GSUB)])
        pltpu.sync_copy(jcomp_v.at[pl.ds(i * GSUB, GSUB)],
                        jidx_hbm.at[pl.ds(dst0, GSUB)])
        return jnp.int32(0)

    lax.fori_loop(0, nwin, win, jnp.int32(0))


def _cmlp_body(cnt_ref, e_ref, embb_ref, w1_ref, b1_ref, w2_ref, b2_ref,
               out_ref):
    s = pl.program_id(0)
    i = pl.program_id(1)

    @pl.when(i * TILE_M < cnt_ref[s])
    def _():
        h = e_ref[...] + embb_ref[...]
        t = _elu(jnp.dot(h, w1_ref[...], preferred_element_type=jnp.float32)
                 + b1_ref[...])
        res = _elu(jnp.dot(t, w2_ref[...], preferred_element_type=jnp.float32)
                   + b2_ref[...])
        out_ref[...] = _elu(h + res)


def _clamp_tile(i, cnt):
    n_act = (cnt + TILE_M - 1) // TILE_M
    return jnp.minimum(i, jnp.maximum(n_act - 1, 0))


def _sc_scatter_body(cnts_hbm, jidx_hbm, cres_hbm, out_ref,
                     cnts_v, idx_v, dst_v, rows_v, sem):
    cid = lax.axis_index("c")
    sid = lax.axis_index("s")
    pltpu.sync_copy(cnts_hbm, cnts_v)

    off = cid * HALF
    cnt = jnp.int32(0)
    for t in range(NS):
        c_t = cnts_v[cid * NS + t, pl.ds(0, L)][0]
        off = off + jnp.where(t < sid, _rup128(c_t), 0)
        cnt = cnt + jnp.where(t == sid, c_t, 0)
    nwin = (cnt + GSUB - 1) // GSUB

    def win(i, _):
        src0 = pl.multiple_of(off + i * GSUB, 8)
        pltpu.sync_copy(jidx_hbm.at[pl.ds(src0, GSUB)], idx_v)
        pltpu.sync_copy(cres_hbm.at[pl.ds(src0, GSUB)], rows_v)
        for t in range(GSUB // L):
            dst_v[pl.ds(t * L, L)] = idx_v[pl.ds(t * L, L)] + N_ATTR
        pltpu.async_copy(rows_v, out_ref.at[dst_v], sem).wait()
        return jnp.int32(0)

    lax.fori_loop(0, nwin, win, jnp.int32(0))


@jax.jit
def kernel(x_attr, node_assign, W_pre, b_pre, emb_W, emb_b, W_ops, b_ops,
           W_res1, b_res1, W_res2, b_res2):
    a32 = node_assign.astype(jnp.int32)
    a_u = jnp.pad(a32[N_ATTR:], (0, N_UN_PAD - N_UN), constant_values=1)
    a_all = a32.reshape(N_TOTAL // TILE_D, 1, TILE_D)
    b_pre2 = b_pre.reshape(1, D_HID)
    emb_b2 = emb_b.reshape(1, D_HID)
    b1_2 = b_res1.reshape(1, 2 * D_HID)
    b2_2 = b_res2.reshape(1, D_HID)

    const_spec = lambda shape: pl.BlockSpec(shape, lambda *_: (0,) * len(shape))

    tbl = pl.pallas_call(
        _table_body,
        out_shape=jax.ShapeDtypeStruct((K, D_HID), jnp.float32),
    )(b_ops, W_res1, b1_2, W_res2, b2_2)

    out_dense = pl.pallas_call(
        _dense_body,
        grid=(N_TOTAL // TILE_D,),
        in_specs=[
            pl.BlockSpec((1, 1, TILE_D), lambda i: (i, 0, 0)),
            pl.BlockSpec((TILE_D, D_IN),
                         lambda i: (jnp.minimum(i, N_ATTR // TILE_D - 1), 0)),
            const_spec((K, D_HID)),
            const_spec((D_IN, D_HID)),
            const_spec((1, D_HID)),
            const_spec((K - 1, D_HID, D_HID)),
            const_spec((K - 1, D_HID)),
            const_spec((D_HID, 2 * D_HID)),
            const_spec((1, 2 * D_HID)),
            const_spec((2 * D_HID, D_HID)),
            const_spec((1, D_HID)),
        ],
        out_specs=pl.BlockSpec((TILE_D, D_HID), lambda i: (i, 0)),
        out_shape=jax.ShapeDtypeStruct((N_TOTAL, D_HID), jnp.float32),
    )(a_all, x_attr, tbl, W_pre, b_pre2, W_ops, b_ops, W_res1, b1_2,
      W_res2, b2_2)

    mesh = plsc.VectorSubcoreMesh(core_axis_name="c", subcore_axis_name="s")

    sc_route = functools.partial(
        pl.kernel, mesh=mesh,
        compiler_params=pltpu.CompilerParams(needs_layout_passes=False),
        out_type=[
            jax.ShapeDtypeStruct((CBUF_ROWS, D_HID), jnp.float32),
            jax.ShapeDtypeStruct((CAP,), jnp.int32),
            jax.ShapeDtypeStruct((NC * NS * L,), jnp.int32),
            jax.ShapeDtypeStruct((NC * L,), jnp.int32),
        ],
        scratch_types=[
            pltpu.VMEM((HALF + L,), jnp.int32),
            pltpu.VMEM((CHUNK + L,), jnp.int32),
            pltpu.VMEM((GSUB, D_HID), jnp.float32),
            pltpu.VMEM((L,), jnp.int32),
            pltpu.SemaphoreType.DMA,
        ],
    )(_sc_route_body)
    gath, jidx, cnts, tot32 = sc_route(a_u, emb_W)

    cnt2 = jnp.stack([tot32[0], tot32[L]])

    grid_spec = pltpu.PrefetchScalarGridSpec(
        num_scalar_prefetch=1,
        grid=(NC, HALF // TILE_M),
        in_specs=[
            pl.BlockSpec(
                (TILE_M, D_HID),
                lambda s, i, c: (s * (HALF // TILE_M) + _clamp_tile(i, c[s]), 0)),
            pl.BlockSpec((1, D_HID), lambda s, i, c: (0, 0)),
            pl.BlockSpec((D_HID, 2 * D_HID), lambda s, i, c: (0, 0)),
            pl.BlockSpec((1, 2 * D_HID), lambda s, i, c: (0, 0)),
            pl.BlockSpec((2 * D_HID, D_HID), lambda s, i, c: (0, 0)),
            pl.BlockSpec((1, D_HID), lambda s, i, c: (0, 0)),
        ],
        out_specs=pl.BlockSpec(
            (TILE_M, D_HID),
            lambda s, i, c: (s * (HALF // TILE_M) + _clamp_tile(i, c[s]), 0)),
    )
    cres = pl.pallas_call(
        _cmlp_body,
        grid_spec=grid_spec,
        out_shape=jax.ShapeDtypeStruct((CBUF_ROWS, D_HID), jnp.float32),
    )(cnt2, gath, emb_b2, W_res1, b1_2, W_res2, b2_2)

    cnts2d = cnts.reshape(NC * NS, L)

    sc_scatter = functools.partial(
        pl.kernel, mesh=mesh,
        compiler_params=pltpu.CompilerParams(needs_layout_passes=False),
        out_type=(),
        scratch_types=[
            pltpu.VMEM((NC * NS, L), jnp.int32),
            pltpu.VMEM((GSUB,), jnp.int32),
            pltpu.VMEM((GSUB,), jnp.int32),
            pltpu.VMEM((GSUB, D_HID), jnp.float32),
            pltpu.SemaphoreType.DMA,
        ],
    )(_sc_scatter_body)

    o_ref = jax.new_ref(out_dense)
    sc_scatter(cnts2d, jidx, cres, o_ref)
    return o_ref[...]


# speedup vs baseline: 1.0141x; 1.0141x over previous
import functools

import jax
import jax.numpy as jnp
from jax import lax
from jax.experimental import pallas as pl
from jax.experimental.pallas import tpu as pltpu
from jax.experimental.pallas import tpu_sc as plsc

N_TOTAL = 50000
N_ATTR = 10000
N_UN = N_TOTAL - N_ATTR
D_IN = 512
D_HID = 256
K = 8

TILE_D = 2000
NC, NS, L = 2, 16, 16
CHUNK = 1280
N_UN_PAD = NC * NS * CHUNK
HALF = NS * CHUNK
GSUB = 128
NVR = CHUNK // L
TILE_M = 2048
CAP = NC * HALF
CBUF_ROWS = 43008


def _elu(x):
    return jnp.where(x > 0, x, jnp.exp(jnp.minimum(x, 0.0)) - 1.0)


def _rup128(c):
    return ((c + GSUB - 1) // GSUB) * GSUB


def _table_body(bops_ref, w1_ref, b1_ref, w2_ref, b2_ref, out_ref):
    c = jnp.concatenate(
        [jnp.zeros((1, D_HID), jnp.float32), _elu(bops_ref[...])], axis=0)
    t = _elu(jnp.dot(c, w1_ref[...], preferred_element_type=jnp.float32)
             + b1_ref[...])
    res = _elu(jnp.dot(t, w2_ref[...], preferred_element_type=jnp.float32)
               + b2_ref[...])
    out_ref[...] = _elu(c + res)


def _dense_body(a_ref, x_ref, tbl_ref, wpre_ref, bpre_ref, wops_ref,
                bops_ref, w1_ref, b1_ref, w2_ref, b2_ref, out_ref):
    i = pl.program_id(0)
    a = a_ref[0, 0, :][:, None]

    @pl.when(i < N_ATTR // TILE_D)
    def _():
        x = x_ref[...]
        h_tr = jnp.dot(x, wpre_ref[...], preferred_element_type=jnp.float32)
        h_tr = h_tr + bpre_ref[...]
        acc = jnp.zeros((TILE_D, D_HID), dtype=jnp.float32)
        for k in range(1, K):
            o = jnp.dot(h_tr, wops_ref[k - 1],
                        preferred_element_type=jnp.float32)
            o = _elu(o + bops_ref[k - 1][None, :])
            acc = acc + jnp.where(a == k, o, 0.0)
        t = _elu(jnp.dot(acc, w1_ref[...], preferred_element_type=jnp.float32)
                 + b1_ref[...])
        res = _elu(jnp.dot(t, w2_ref[...], preferred_element_type=jnp.float32)
                   + b2_ref[...])
        out_ref[...] = _elu(acc + res) + h_tr

    @pl.when(i >= N_ATTR // TILE_D)
    def _():
        acc = jnp.zeros((TILE_D, D_HID), dtype=jnp.float32)
        for k in range(K):
            acc = acc + jnp.where(a == k, tbl_ref[k][None, :], 0.0)
        out_ref[...] = acc


def _sc_route_body(a_hbm, emb_hbm, gath_hbm, jidx_hbm, cnts_hbm, tot_hbm,
                   half_v, jcomp_v, rows_v, cbuf_v, sem):
    cid = lax.axis_index("c")
    sid = lax.axis_index("s")
    hbase = pl.multiple_of(cid * HALF, 8)
    pltpu.sync_copy(a_hbm.at[pl.ds(hbase, HALF)], half_v.at[pl.ds(0, HALF)])
    lane = lax.broadcasted_iota(jnp.int32, (L,), 0)

    cnt_t = []
    for t in range(NS):
        def step(v, acc):
            av = half_v[pl.ds(v * L, L)]
            return acc + (av == 0).astype(jnp.int32)
        acc = lax.fori_loop(t * NVR, (t + 1) * NVR, step,
                            jnp.zeros((L,), jnp.int32))
        cnt_t.append(jnp.cumsum(acc)[L - 1])

    off = cid * HALF
    cnt = jnp.int32(0)
    tot = jnp.int32(0)
    for t in range(NS):
        off = off + jnp.where(t < sid, _rup128(cnt_t[t]), 0)
        cnt = cnt + jnp.where(t == sid, cnt_t[t], 0)
        tot = tot + _rup128(cnt_t[t])
    base = cid * HALF + sid * CHUNK

    cbuf_v[...] = (lane == 0).astype(jnp.int32) * cnt
    w16 = pl.multiple_of((cid * NS + sid) * L, 8)
    pltpu.sync_copy(cbuf_v, cnts_hbm.at[pl.ds(w16, L)])

    @pl.when(sid == 0)
    def _():
        cbuf_v[...] = (lane == 0).astype(jnp.int32) * tot
        pltpu.sync_copy(cbuf_v, tot_hbm.at[pl.ds(pl.multiple_of(cid * L, 8), L)])

    zero16 = jnp.zeros((L,), jnp.int32)
    for v in range(NVR):
        jcomp_v[pl.ds(v * L, L)] = zero16
    run = jnp.int32(0)
    for v in range(NVR):
        av = half_v[pl.ds(sid * CHUNK + v * L, L)]
        m = av == 0
        mi = m.astype(jnp.int32)
        inc = jnp.cumsum(mi)
        jvec = base + v * L + lane
        plsc.store_scatter(jcomp_v, [run + inc - mi], jvec, mask=m)
        run = run + inc[L - 1]

    @pl.when(cnt > 0)
    def _():
        lastv = jcomp_v[pl.ds(cnt - 1, L)][0]
        for v in range(NVR):
            cv = jcomp_v[pl.ds(v * L, L)]
            g = v * L + lane
            jcomp_v[pl.ds(v * L, L)] = jnp.where(g < cnt, cv, lastv)

    nwin = (cnt + GSUB - 1) // GSUB

    def win(i, _):
        pltpu.async_copy(
            emb_hbm.at[jcomp_v.at[pl.ds(i * GSUB, GSUB)]], rows_v, sem
        ).wait()
        dst0 = pl.multiple_of(off + i * GSUB, 8)
        pltpu.sync_copy(rows_v, gath_hbm.at[pl.ds(dst0, GSUB)])
        pltpu.sync_copy(jcomp_v.at[pl.ds(i * GSUB, GSUB)],
                        jidx_hbm.at[pl.ds(dst0, GSUB)])
        return jnp.int32(0)

    lax.fori_loop(0, nwin, win, jnp.int32(0))


def _cmlp_body(cnt_ref, e_ref, embb_ref, w1_ref, b1_ref, w2_ref, b2_ref,
               out_ref):
    s = pl.program_id(0)
    i = pl.program_id(1)

    @pl.when(i * TILE_M < cnt_ref[s])
    def _():
        h = e_ref[...] + embb_ref[...]
        t = _elu(jnp.dot(h, w1_ref[...], preferred_element_type=jnp.float32)
                 + b1_ref[...])
        res = _elu(jnp.dot(t, w2_ref[...], preferred_element_type=jnp.float32)
                   + b2_ref[...])
        out_ref[...] = _elu(h + res)


def _clamp_tile(i, cnt):
    n_act = (cnt + TILE_M - 1) // TILE_M
    return jnp.minimum(i, jnp.maximum(n_act - 1, 0))


def _sc_scatter_body(cnts_hbm, jidx_hbm, cres_hbm, out_ref,
                     cnts_v, idx_v, dst_v, rows_v, sem):
    cid = lax.axis_index("c")
    sid = lax.axis_index("s")
    pltpu.sync_copy(cnts_hbm, cnts_v)

    off = cid * HALF
    cnt = jnp.int32(0)
    for t in range(NS):
        c_t = cnts_v[cid * NS + t, pl.ds(0, L)][0]
        off = off + jnp.where(t < sid, _rup128(c_t), 0)
        cnt = cnt + jnp.where(t == sid, c_t, 0)
    nwin = (cnt + GSUB - 1) // GSUB

    def win(i, _):
        src0 = pl.multiple_of(off + i * GSUB, 8)
        pltpu.sync_copy(jidx_hbm.at[pl.ds(src0, GSUB)], idx_v)
        pltpu.sync_copy(cres_hbm.at[pl.ds(src0, GSUB)], rows_v)
        for t in range(GSUB // L):
            dst_v[pl.ds(t * L, L)] = idx_v[pl.ds(t * L, L)] + N_ATTR
        pltpu.async_copy(rows_v, out_ref.at[dst_v], sem).wait()
        return jnp.int32(0)

    lax.fori_loop(0, nwin, win, jnp.int32(0))


@jax.jit
def kernel(x_attr, node_assign, W_pre, b_pre, emb_W, emb_b, W_ops, b_ops,
           W_res1, b_res1, W_res2, b_res2):
    a32 = node_assign.astype(jnp.int32)
    a_u = jnp.pad(a32[N_ATTR:], (0, N_UN_PAD - N_UN), constant_values=1)
    a_all = a32.reshape(N_TOTAL // TILE_D, 1, TILE_D)
    b_pre2 = b_pre.reshape(1, D_HID)
    emb_b2 = emb_b.reshape(1, D_HID)
    b1_2 = b_res1.reshape(1, 2 * D_HID)
    b2_2 = b_res2.reshape(1, D_HID)

    const_spec = lambda shape: pl.BlockSpec(shape, lambda *_: (0,) * len(shape))

    tbl = pl.pallas_call(
        _table_body,
        out_shape=jax.ShapeDtypeStruct((K, D_HID), jnp.float32),
    )(b_ops, W_res1, b1_2, W_res2, b2_2)

    out_dense = pl.pallas_call(
        _dense_body,
        grid=(N_TOTAL // TILE_D,),
        in_specs=[
            pl.BlockSpec((1, 1, TILE_D), lambda i: (i, 0, 0)),
            pl.BlockSpec((TILE_D, D_IN),
                         lambda i: (jnp.minimum(i, N_ATTR // TILE_D - 1), 0)),
            const_spec((K, D_HID)),
            const_spec((D_IN, D_HID)),
            const_spec((1, D_HID)),
            const_spec((K - 1, D_HID, D_HID)),
            const_spec((K - 1, D_HID)),
            const_spec((D_HID, 2 * D_HID)),
            const_spec((1, 2 * D_HID)),
            const_spec((2 * D_HID, D_HID)),
            const_spec((1, D_HID)),
        ],
        out_specs=pl.BlockSpec((TILE_D, D_HID), lambda i: (i, 0)),
        out_shape=jax.ShapeDtypeStruct((N_TOTAL, D_HID), jnp.float32),
    )(a_all, x_attr, tbl, W_pre, b_pre2, W_ops, b_ops, W_res1, b1_2,
      W_res2, b2_2)

    mesh = plsc.VectorSubcoreMesh(core_axis_name="c", subcore_axis_name="s")

    sc_route = functools.partial(
        pl.kernel, mesh=mesh,
        compiler_params=pltpu.CompilerParams(needs_layout_passes=False),
        out_type=[
            jax.ShapeDtypeStruct((CBUF_ROWS, D_HID), jnp.float32),
            jax.ShapeDtypeStruct((CAP,), jnp.int32),
            jax.ShapeDtypeStruct((NC * NS * L,), jnp.int32),
            jax.ShapeDtypeStruct((NC * L,), jnp.int32),
        ],
        scratch_types=[
            pltpu.VMEM((HALF + L,), jnp.int32),
            pltpu.VMEM((CHUNK + L,), jnp.int32),
            pltpu.VMEM((GSUB, D_HID), jnp.float32),
            pltpu.VMEM((L,), jnp.int32),
            pltpu.SemaphoreType.DMA,
        ],
    )(_sc_route_body)
    gath, jidx, cnts, tot32 = sc_route(a_u, emb_W)

    cnt2 = jnp.stack([tot32[0], tot32[L]])

    grid_spec = pltpu.PrefetchScalarGridSpec(
        num_scalar_prefetch=1,
        grid=(NC, HALF // TILE_M),
        in_specs=[
            pl.BlockSpec(
                (TILE_M, D_HID),
                lambda s, i, c: (s * (HALF // TILE_M) + _clamp_tile(i, c[s]), 0)),
            pl.BlockSpec((1, D_HID), lambda s, i, c: (0, 0)),
            pl.BlockSpec((D_HID, 2 * D_HID), lambda s, i, c: (0, 0)),
            pl.BlockSpec((1, 2 * D_HID), lambda s, i, c: (0, 0)),
            pl.BlockSpec((2 * D_HID, D_HID), lambda s, i, c: (0, 0)),
            pl.BlockSpec((1, D_HID), lambda s, i, c: (0, 0)),
        ],
        out_specs=pl.BlockSpec(
            (TILE_M, D_HID),
            lambda s, i, c: (s * (HALF // TILE_M) + _clamp_tile(i, c[s]), 0)),
    )
    cres = pl.pallas_call(
        _cmlp_body,
        grid_spec=grid_spec,
        out_shape=jax.ShapeDtypeStruct((CBUF_ROWS, D_HID), jnp.float32),
    )(cnt2, gath, emb_b2, W_res1, b1_2, W_res2, b2_2)

    cnts2d = cnts.reshape(NC * NS, L)

    sc_scatter = functools.partial(
        pl.kernel, mesh=mesh,
        compiler_params=pltpu.CompilerParams(needs_layout_passes=False),
        out_type=(),
        scratch_types=[
            pltpu.VMEM((NC * NS, L), jnp.int32),
            pltpu.VMEM((GSUB,), jnp.int32),
            pltpu.VMEM((GSUB,), jnp.int32),
            pltpu.VMEM((GSUB, D_HID), jnp.float32),
            pltpu.SemaphoreType.DMA,
        ],
    )(_sc_scatter_body)

    o_ref = jax.new_ref(out_dense)
    sc_scatter(cnts2d, jidx, cres, o_ref)
    return o_ref[...]
